# baseline (speedup 1.0000x reference)
import functools

import jax
import jax.numpy as jnp
from jax import lax
from jax.experimental import pallas as pl
from jax.experimental.pallas import tpu as pltpu
from jax.experimental.pallas import tpu_sc as plsc

B = 8
H = 224
W = 224
HW = H * W
N_M = 4096
N_UP = 1024
N_UT = 512
LANES = 16
NMQ = N_M // 4
NUPQ = N_UP // 4
NUTQ = N_UT // 4
O_PMD = NMQ
O_TMB = 2 * NMQ
O_TMD = 3 * NMQ
O_PUB = 4 * NMQ
O_PUD = O_PUB + NUPQ
O_TUB = O_PUD + NUPQ
O_TUD = O_TUB + NUTQ
PER_TILE = O_TUD + NUTQ
N_ALL = 4 * PER_TILE
M_IT = NMQ // LANES
UP_IT = NUPQ // LANES
UT_IT = NUTQ // LANES


def _sc_loss_kernel(input_hbm, target_hbm, coords_hbm, out_hbm,
                    fp_v, ft_v, idxr_v, idxc_v,
                    part_v, red_v, out_v, sem,
                    partials_sp):
    c = lax.axis_index("c")
    s = lax.axis_index("s")
    q = s % 4
    b = c * 4 + s // 4

    o = q * PER_TILE
    cp_r = pltpu.async_copy(coords_hbm.at[b, 0, pl.ds(o, PER_TILE)],
                            idxr_v, sem)
    cp_c = pltpu.async_copy(coords_hbm.at[b, 1, pl.ds(o, PER_TILE)],
                            idxc_v, sem)
    cp_fp = pltpu.async_copy(input_hbm.at[b, 0], fp_v, sem)
    cp_ft = pltpu.async_copy(target_hbm.at[b, 0], ft_v, sem)
    cp_r.wait()
    cp_c.wait()
    cp_fp.wait()

    def sig(v):
        return 1.0 / (1.0 + jnp.exp(-v))

    def g(field, off):
        return plsc.load_gather(
            field, [idxr_v[pl.ds(off, LANES)], idxc_v[pl.ds(off, LANES)]])

    def up_body(i, acc):
        o = i * LANES
        d = sig(g(fp_v, O_PUB + o)) - sig(g(fp_v, O_PUD + o))
        return acc + d * d
    acc0 = lax.fori_loop(0, UP_IT, up_body,
                         jnp.zeros((LANES,), jnp.float32), unroll=2)

    cp_ft.wait()

    def matched_body(i, acc):
        o = i * LANES
        db = sig(g(fp_v, o)) - g(ft_v, O_TMB + o)
        dd = sig(g(fp_v, O_PMD + o)) - g(ft_v, O_TMD + o)
        return acc + (db * db + dd * dd)
    acc = lax.fori_loop(0, M_IT, matched_body,
                        jnp.zeros((LANES,), jnp.float32), unroll=2)
    acc = acc + acc

    def ut_body(i, acc):
        o = i * LANES
        d = g(ft_v, O_TUB + o) - g(ft_v, O_TUD + o)
        return acc + d * d
    acc = lax.fori_loop(0, UT_IT, ut_body, acc + acc0, unroll=2)

    part_v[...] = acc
    pltpu.sync_copy(part_v, partials_sp.at[pl.ds(s * LANES, LANES)])
    plsc.subcore_barrier()

    @pl.when(s == 0)
    def _reduce():
        pltpu.sync_copy(partials_sp, red_v)

        def body(j, acc):
            return acc + red_v[pl.ds(j * LANES, LANES)]
        tot = lax.fori_loop(0, 16, body, jnp.zeros((LANES,), jnp.float32))
        total = jnp.sum(tot) * jnp.float32(1.0 / B)
        out_v[...] = jnp.broadcast_to(total, (LANES,))
        pltpu.sync_copy(out_v, out_hbm.at[c])


_sc_loss = functools.partial(
    pl.kernel,
    mesh=plsc.VectorSubcoreMesh(core_axis_name="c", subcore_axis_name="s"),
    out_type=jax.ShapeDtypeStruct((2, LANES), jnp.float32),
    compiler_params=pltpu.CompilerParams(needs_layout_passes=False),
    scratch_types=[
        pltpu.VMEM((H, W), jnp.float32),
        pltpu.VMEM((H, W), jnp.float32),
        pltpu.VMEM((PER_TILE,), jnp.int32),
        pltpu.VMEM((PER_TILE,), jnp.int32),
        pltpu.VMEM((LANES,), jnp.float32),
        pltpu.VMEM((16 * LANES,), jnp.float32),
        pltpu.VMEM((LANES,), jnp.float32),
        pltpu.SemaphoreType.DMA,
        pltpu.VMEM_SHARED((16 * LANES,), jnp.float32),
    ],
)(_sc_loss_kernel)


@jax.jit
def kernel(input, target, pred_mb, pred_md, tgt_mb, tgt_md,
           pred_ub, pred_ud, tgt_ub, tgt_ud):
    parts = []
    for q in range(4):
        def qs(x, n):
            return x[:, q * n:(q + 1) * n]
        parts += [qs(pred_mb, NMQ), qs(pred_md, NMQ),
                  qs(tgt_mb, NMQ), qs(tgt_md, NMQ),
                  qs(pred_ub, NUPQ), qs(pred_ud, NUPQ),
                  qs(tgt_ub, NUTQ), qs(tgt_ud, NUTQ)]
    coords = jnp.concatenate(parts, axis=1).astype(jnp.int32)
    coords_t = coords.transpose(0, 2, 1)
    out = _sc_loss(input, target, coords_t)
    return out[0, 0] + out[1, 0]


# speedup vs baseline: 1.3117x; 1.3117x over previous
import functools

import jax
import jax.numpy as jnp
from jax import lax
from jax.experimental import pallas as pl
from jax.experimental.pallas import tpu as pltpu
from jax.experimental.pallas import tpu_sc as plsc

B = 8
H = 224
W = 224
HW = H * W
N_M = 4096
N_UP = 1024
N_UT = 512
LANES = 16
NMQ = N_M // 4
NUPQ = N_UP // 4
NUTQ = N_UT // 4
O_PMB = 0
O_PMD = N_M
O_TMB = 2 * N_M
O_TMD = 3 * N_M
O_PUB = 4 * N_M
O_PUD = O_PUB + N_UP
O_TUB = O_PUD + N_UP
O_TUD = O_TUB + N_UT
N_ALL = O_TUD + N_UT
M_IT = NMQ // LANES
UP_IT = NUPQ // LANES
UT_IT = NUTQ // LANES


def _sc_loss_kernel(input_hbm, target_hbm, coords_hbm, out_hbm,
                    fp_v, ft_v,
                    mbr_v, mbc_v, mdr_v, mdc_v,
                    tbr_v, tbc_v, tdr_v, tdc_v,
                    ubr_v, ubc_v, udr_v, udc_v,
                    vbr_v, vbc_v, vdr_v, vdc_v,
                    part_v, red_v, out_v, sem,
                    partials_sp):
    c = lax.axis_index("c")
    s = lax.axis_index("s")
    q = s % 4
    b = c * 4 + s // 4

    cps = []
    for off, n, rbuf, cbuf in (
            (O_PMB, NMQ, mbr_v, mbc_v), (O_PMD, NMQ, mdr_v, mdc_v),
            (O_TMB, NMQ, tbr_v, tbc_v), (O_TMD, NMQ, tdr_v, tdc_v),
            (O_PUB, NUPQ, ubr_v, ubc_v), (O_PUD, NUPQ, udr_v, udc_v),
            (O_TUB, NUTQ, vbr_v, vbc_v), (O_TUD, NUTQ, vdr_v, vdc_v)):
        cps.append(pltpu.async_copy(
            coords_hbm.at[b, 0, pl.ds(off + q * n, n)], rbuf, sem))
        cps.append(pltpu.async_copy(
            coords_hbm.at[b, 1, pl.ds(off + q * n, n)], cbuf, sem))
    cp_fp = pltpu.async_copy(input_hbm.at[b, 0], fp_v, sem)
    cp_ft = pltpu.async_copy(target_hbm.at[b, 0], ft_v, sem)
    for cp in cps:
        cp.wait()
    cp_fp.wait()

    def sig(v):
        return 1.0 / (1.0 + jnp.exp(-v))

    def g(field, rbuf, cbuf, o):
        return plsc.load_gather(
            field, [rbuf[pl.ds(o, LANES)], cbuf[pl.ds(o, LANES)]])

    def up_body(i, acc):
        o = i * LANES
        d = sig(g(fp_v, ubr_v, ubc_v, o)) - sig(g(fp_v, udr_v, udc_v, o))
        return acc + d * d
    acc0 = lax.fori_loop(0, UP_IT, up_body,
                         jnp.zeros((LANES,), jnp.float32), unroll=2)

    cp_ft.wait()

    def matched_body(i, acc):
        o = i * LANES
        db = sig(g(fp_v, mbr_v, mbc_v, o)) - g(ft_v, tbr_v, tbc_v, o)
        dd = sig(g(fp_v, mdr_v, mdc_v, o)) - g(ft_v, tdr_v, tdc_v, o)
        return acc + (db * db + dd * dd)
    acc = lax.fori_loop(0, M_IT, matched_body,
                        jnp.zeros((LANES,), jnp.float32), unroll=4)
    acc = acc + acc

    def ut_body(i, acc):
        o = i * LANES
        d = g(ft_v, vbr_v, vbc_v, o) - g(ft_v, vdr_v, vdc_v, o)
        return acc + d * d
    acc = lax.fori_loop(0, UT_IT, ut_body, acc + acc0, unroll=2)

    part_v[...] = acc
    pltpu.sync_copy(part_v, partials_sp.at[pl.ds(s * LANES, LANES)])
    plsc.subcore_barrier()

    @pl.when(s == 0)
    def _reduce():
        pltpu.sync_copy(partials_sp, red_v)

        def body(j, acc):
            return acc + red_v[pl.ds(j * LANES, LANES)]
        tot = lax.fori_loop(0, 16, body, jnp.zeros((LANES,), jnp.float32))
        total = jnp.sum(tot) * jnp.float32(1.0 / B)
        out_v[...] = jnp.broadcast_to(total, (LANES,))
        pltpu.sync_copy(out_v, out_hbm.at[c])


_sc_loss = functools.partial(
    pl.kernel,
    mesh=plsc.VectorSubcoreMesh(core_axis_name="c", subcore_axis_name="s"),
    out_type=jax.ShapeDtypeStruct((2, LANES), jnp.float32),
    compiler_params=pltpu.CompilerParams(needs_layout_passes=False),
    scratch_types=[
        pltpu.VMEM((H, W), jnp.float32),
        pltpu.VMEM((H, W), jnp.float32),
        pltpu.VMEM((NMQ,), jnp.int32),
        pltpu.VMEM((NMQ,), jnp.int32),
        pltpu.VMEM((NMQ,), jnp.int32),
        pltpu.VMEM((NMQ,), jnp.int32),
        pltpu.VMEM((NMQ,), jnp.int32),
        pltpu.VMEM((NMQ,), jnp.int32),
        pltpu.VMEM((NMQ,), jnp.int32),
        pltpu.VMEM((NMQ,), jnp.int32),
        pltpu.VMEM((NUPQ,), jnp.int32),
        pltpu.VMEM((NUPQ,), jnp.int32),
        pltpu.VMEM((NUPQ,), jnp.int32),
        pltpu.VMEM((NUPQ,), jnp.int32),
        pltpu.VMEM((NUTQ,), jnp.int32),
        pltpu.VMEM((NUTQ,), jnp.int32),
        pltpu.VMEM((NUTQ,), jnp.int32),
        pltpu.VMEM((NUTQ,), jnp.int32),
        pltpu.VMEM((LANES,), jnp.float32),
        pltpu.VMEM((16 * LANES,), jnp.float32),
        pltpu.VMEM((LANES,), jnp.float32),
        pltpu.SemaphoreType.DMA,
        pltpu.VMEM_SHARED((16 * LANES,), jnp.float32),
    ],
)(_sc_loss_kernel)


@jax.jit
def kernel(input, target, pred_mb, pred_md, tgt_mb, tgt_md,
           pred_ub, pred_ud, tgt_ub, tgt_ud):
    coords = jnp.concatenate(
        [pred_mb, pred_md, tgt_mb, tgt_md,
         pred_ub, pred_ud, tgt_ub, tgt_ud], axis=1).astype(jnp.int32)
    coords_t = coords.transpose(0, 2, 1)
    out = _sc_loss(input, target, coords_t)
    return out[0, 0] + out[1, 0]


# speedup vs baseline: 1.3150x; 1.0025x over previous
import functools

import jax
import jax.numpy as jnp
from jax import lax
from jax.experimental import pallas as pl
from jax.experimental.pallas import tpu as pltpu
from jax.experimental.pallas import tpu_sc as plsc

B = 8
H = 224
W = 224
HW = H * W
N_M = 4096
N_UP = 1024
N_UT = 512
LANES = 16
NMQ = N_M // 4
NUPQ = N_UP // 4
NUTQ = N_UT // 4
O_PMB = 0
O_PMD = N_M
O_TMB = 2 * N_M
O_TMD = 3 * N_M
O_PUB = 4 * N_M
O_PUD = O_PUB + N_UP
O_TUB = O_PUD + N_UP
O_TUD = O_TUB + N_UT
N_ALL = O_TUD + N_UT
M_IT = NMQ // LANES
UP_IT = NUPQ // LANES
UT_IT = NUTQ // LANES


def _sc_loss_kernel(input_hbm, target_hbm, coords_hbm, out_hbm,
                    fp_v, ft_v,
                    mbr_v, mbc_v, mdr_v, mdc_v,
                    tbr_v, tbc_v, tdr_v, tdc_v,
                    ubr_v, ubc_v, udr_v, udc_v,
                    vbr_v, vbc_v, vdr_v, vdc_v,
                    part_v, red_v, out_v, sem,
                    partials_sp):
    c = lax.axis_index("c")
    s = lax.axis_index("s")
    q = s % 4
    b = c * 4 + s // 4

    cps = []
    for off, n, rbuf, cbuf in (
            (O_PMB, NMQ, mbr_v, mbc_v), (O_PMD, NMQ, mdr_v, mdc_v),
            (O_TMB, NMQ, tbr_v, tbc_v), (O_TMD, NMQ, tdr_v, tdc_v),
            (O_PUB, NUPQ, ubr_v, ubc_v), (O_PUD, NUPQ, udr_v, udc_v),
            (O_TUB, NUTQ, vbr_v, vbc_v), (O_TUD, NUTQ, vdr_v, vdc_v)):
        cps.append(pltpu.async_copy(
            coords_hbm.at[b, 0, pl.ds(off + q * n, n)], rbuf, sem))
        cps.append(pltpu.async_copy(
            coords_hbm.at[b, 1, pl.ds(off + q * n, n)], cbuf, sem))
    cp_fp = pltpu.async_copy(input_hbm.at[b, 0], fp_v, sem)
    cp_ft = pltpu.async_copy(target_hbm.at[b, 0], ft_v, sem)
    for cp in cps:
        cp.wait()
    cp_fp.wait()

    def sig(v):
        return 1.0 / (1.0 + jnp.exp(-v))

    def g(field, rbuf, cbuf, o):
        return plsc.load_gather(
            field, [rbuf[pl.ds(o, LANES)], cbuf[pl.ds(o, LANES)]])

    def up_body(i, acc):
        o = i * LANES
        d = sig(g(fp_v, ubr_v, ubc_v, o)) - sig(g(fp_v, udr_v, udc_v, o))
        return acc + d * d
    acc0 = lax.fori_loop(0, UP_IT, up_body,
                         jnp.zeros((LANES,), jnp.float32), unroll=2)

    cp_ft.wait()

    def matched_body(i, acc):
        o = i * LANES
        db = sig(g(fp_v, mbr_v, mbc_v, o)) - g(ft_v, tbr_v, tbc_v, o)
        dd = sig(g(fp_v, mdr_v, mdc_v, o)) - g(ft_v, tdr_v, tdc_v, o)
        return acc + (db * db + dd * dd)
    acc = lax.fori_loop(0, M_IT, matched_body,
                        jnp.zeros((LANES,), jnp.float32), unroll=4)
    acc = acc + acc

    def ut_body(i, acc):
        o = i * LANES
        d = g(ft_v, vbr_v, vbc_v, o) - g(ft_v, vdr_v, vdc_v, o)
        return acc + d * d
    acc = lax.fori_loop(0, UT_IT, ut_body, acc + acc0, unroll=2)

    part_v[...] = acc
    pltpu.sync_copy(part_v, partials_sp.at[pl.ds(s * LANES, LANES)])
    plsc.subcore_barrier()

    @pl.when(s == 0)
    def _reduce():
        pltpu.sync_copy(partials_sp, red_v)

        def body(j, acc):
            return acc + red_v[pl.ds(j * LANES, LANES)]
        tot = lax.fori_loop(0, 16, body, jnp.zeros((LANES,), jnp.float32))
        total = jnp.sum(tot) * jnp.float32(1.0 / B)
        out_v[...] = jnp.broadcast_to(total, (LANES,))
        pltpu.sync_copy(out_v, out_hbm.at[c])


_sc_loss = functools.partial(
    pl.kernel,
    mesh=plsc.VectorSubcoreMesh(core_axis_name="c", subcore_axis_name="s"),
    out_type=jax.ShapeDtypeStruct((2, LANES), jnp.float32),
    compiler_params=pltpu.CompilerParams(needs_layout_passes=False, use_tc_tiling_on_sc=True),
    scratch_types=[
        pltpu.VMEM((H, W), jnp.float32),
        pltpu.VMEM((H, W), jnp.float32),
        pltpu.VMEM((NMQ,), jnp.int32),
        pltpu.VMEM((NMQ,), jnp.int32),
        pltpu.VMEM((NMQ,), jnp.int32),
        pltpu.VMEM((NMQ,), jnp.int32),
        pltpu.VMEM((NMQ,), jnp.int32),
        pltpu.VMEM((NMQ,), jnp.int32),
        pltpu.VMEM((NMQ,), jnp.int32),
        pltpu.VMEM((NMQ,), jnp.int32),
        pltpu.VMEM((NUPQ,), jnp.int32),
        pltpu.VMEM((NUPQ,), jnp.int32),
        pltpu.VMEM((NUPQ,), jnp.int32),
        pltpu.VMEM((NUPQ,), jnp.int32),
        pltpu.VMEM((NUTQ,), jnp.int32),
        pltpu.VMEM((NUTQ,), jnp.int32),
        pltpu.VMEM((NUTQ,), jnp.int32),
        pltpu.VMEM((NUTQ,), jnp.int32),
        pltpu.VMEM((LANES,), jnp.float32),
        pltpu.VMEM((16 * LANES,), jnp.float32),
        pltpu.VMEM((LANES,), jnp.float32),
        pltpu.SemaphoreType.DMA,
        pltpu.VMEM_SHARED((16 * LANES,), jnp.float32),
    ],
)(_sc_loss_kernel)


@jax.jit
def kernel(input, target, pred_mb, pred_md, tgt_mb, tgt_md,
           pred_ub, pred_ud, tgt_ub, tgt_ud):
    coords = jnp.concatenate(
        [pred_mb, pred_md, tgt_mb, tgt_md,
         pred_ub, pred_ud, tgt_ub, tgt_ud], axis=1).astype(jnp.int32)
    coords_t = coords.transpose(0, 2, 1)
    out = _sc_loss(input, target, coords_t)
    return out[0, 0] + out[1, 0]
